# baseline (speedup 1.0000x reference)
import functools

import jax
import jax.numpy as jnp
from jax import lax
from jax.experimental import pallas as pl
from jax.experimental.pallas import tpu as pltpu
from jax.experimental.pallas import tpu_sc as plsc

STEPS = 288
EMB = 64
NC, NS = 2, 16
NW = NC * NS
CHUNK = 256
NJ = EMB // 16
NBUF = 2


def _sc_body(x_hbm, w_hbm, b_hbm, tod_hbm, dow_hbm, out_hbm,
             xin_v, x0_v, idx1_v, idx2_v, stage_v, wb_v, tod_t, dow_t,
             in_sems, out_sems):
    n = x_hbm.shape[0] // 3
    npw = n // NW
    wid = lax.axis_index("s") * NC + lax.axis_index("c")
    base0 = wid * npw
    niter = npw // CHUNK

    pltpu.sync_copy(w_hbm, wb_v.at[0])
    pltpu.sync_copy(b_hbm, wb_v.at[1])
    pltpu.sync_copy(tod_hbm, tod_t)
    pltpu.sync_copy(dow_hbm, dow_t)
    w_regs = [wb_v[0, pl.ds(j * 16, 16)] for j in range(NJ)]
    b_regs = [wb_v[1, pl.ds(j * 16, 16)] for j in range(NJ)]
    lane = lax.iota(jnp.int32, 16)
    offs = [lane + (j * 16) for j in range(NJ)]
    lane3 = lane * 3

    def issue_in(g, b):
        base = base0 + g * CHUNK
        pltpu.async_copy(x_hbm.at[pl.ds(base * 3, CHUNK * 3)], xin_v.at[b],
                         in_sems.at[b])

    def wait_in(b):
        pltpu.make_async_copy(x_hbm.at[pl.ds(0, CHUNK * 3)], xin_v.at[b],
                              in_sems.at[b]).wait()

    def wait_out(b):
        pltpu.make_async_copy(stage_v.at[b],
                              out_hbm.at[pl.ds(0, CHUNK)],
                              out_sems.at[b]).wait()

    def compute(g, b):
        @plsc.parallel_loop(0, CHUNK // 16, unroll=2)
        def idx_body(i):
            sl = pl.ds(i * 16, 16)
            g0 = lane3 + i * 48
            x0_v[b, sl] = plsc.load_gather(xin_v.at[b], [g0])
            v1 = plsc.load_gather(xin_v.at[b], [g0 + 1])
            idx1_v[b, sl] = (v1 * float(STEPS)).astype(jnp.int32) * EMB
            v2 = plsc.load_gather(xin_v.at[b], [g0 + 2])
            idx2_v[b, sl] = v2.astype(jnp.int32) * EMB

        @plsc.parallel_loop(0, CHUNK, unroll=4)
        def tok_body(t):
            t16 = jax.lax.broadcast_in_dim(t, (16,), ())
            s16 = plsc.load_gather(x0_v.at[b], [t16])
            tb16 = plsc.load_gather(idx1_v.at[b], [t16])
            db16 = plsc.load_gather(idx2_v.at[b], [t16])
            for j in range(NJ):
                stage_v[b, t, pl.ds(j * 16, 16)] = (
                    s16 * w_regs[j] + b_regs[j])
            for j in range(NJ):
                stage_v[b, t, pl.ds(EMB + j * 16, 16)] = plsc.load_gather(
                    tod_t, [tb16 + offs[j]])
            for j in range(NJ):
                stage_v[b, t, pl.ds(2 * EMB + j * 16, 16)] = plsc.load_gather(
                    dow_t, [db16 + offs[j]])

    def issue_out(g, b):
        base = base0 + g * CHUNK
        pltpu.async_copy(stage_v.at[b], out_hbm.at[pl.ds(base, CHUNK)],
                         out_sems.at[b])

    issue_in(0, 0)

    def pair_body(h, _):
        for b in range(NBUF):
            g = NBUF * h + b
            nb = (b + 1) % NBUF

            @pl.when(g + 1 < niter)
            def _():
                issue_in(g + 1, nb)
            wait_in(b)

            @pl.when(h > 0)
            def _():
                wait_out(b)
            compute(g, b)
            issue_out(g, b)
        return 0

    lax.fori_loop(0, niter // NBUF, pair_body, 0)
    for b in range(NBUF):
        wait_out(b)


@jax.jit
def kernel(x, W_feat, b_feat, tod_table, dow_table):
    B, T, C = x.shape
    n = B * T
    xflat = x.reshape(n * C)
    mesh = plsc.VectorSubcoreMesh(core_axis_name="c", subcore_axis_name="s")
    f = pl.kernel(
        _sc_body,
        out_type=jax.ShapeDtypeStruct((n, 3 * EMB), jnp.float32),
        mesh=mesh,
        compiler_params=pltpu.CompilerParams(
            use_tc_tiling_on_sc=False, needs_layout_passes=False),
        scratch_types=[
            pltpu.VMEM((NBUF, CHUNK * 3), jnp.float32),
            pltpu.VMEM((NBUF, CHUNK), jnp.float32),
            pltpu.VMEM((NBUF, CHUNK), jnp.int32),
            pltpu.VMEM((NBUF, CHUNK), jnp.int32),
            pltpu.VMEM((NBUF, CHUNK, 3 * EMB), jnp.float32),
            pltpu.VMEM((2, EMB), jnp.float32),
            pltpu.VMEM((STEPS * EMB,), jnp.float32),
            pltpu.VMEM((7 * EMB,), jnp.float32),
            pltpu.SemaphoreType.DMA((NBUF,)),
            pltpu.SemaphoreType.DMA((NBUF,)),
        ],
    )
    out = f(xflat, W_feat.reshape(EMB), b_feat,
            tod_table.reshape(STEPS * EMB), dow_table.reshape(7 * EMB))
    return out.reshape(B, T, 3 * EMB)


# speedup vs baseline: 4.0294x; 4.0294x over previous
import functools

import jax
import jax.numpy as jnp
from jax import lax
from jax.experimental import pallas as pl
from jax.experimental.pallas import tpu as pltpu
from jax.experimental.pallas import tpu_sc as plsc

STEPS = 288
EMB = 64
NC, NS = 2, 16
NW = NC * NS
T = 200
B = 16384
BW = B // NW
HW = BW // 2
NK = BW // 16
NK2 = HW // 16
NBUF = 2


def _sc_body(x0_hbm, x1_hbm, x2_hbm, w_hbm, b_hbm, tod_hbm, dow_hbm, out_hbm,
             x0_a, x0_b, x1_a, x1_b, x2_a, x2_b, idx1_v, idx2_v,
             feat_s, tod_s, dow_s, wb_v, tod_t, dow_t, in_sems, out_sems):
    xbufs = ((x0_a, x1_a, x2_a), (x0_b, x1_b, x2_b))
    wid = lax.axis_index("s") * NC + lax.axis_index("c")
    b0 = wid * BW

    pltpu.sync_copy(w_hbm, wb_v.at[pl.ds(0, EMB)])
    pltpu.sync_copy(b_hbm, wb_v.at[pl.ds(EMB, EMB)])
    pltpu.sync_copy(tod_hbm, tod_t)
    pltpu.sync_copy(dow_hbm, dow_t)

    def issue_in(t, b):
        base = t * B + b0
        bx0, bx1, bx2 = xbufs[b]
        pltpu.async_copy(x0_hbm.at[pl.ds(base, BW)], bx0, in_sems.at[b])
        pltpu.async_copy(x1_hbm.at[pl.ds(base, BW)], bx1, in_sems.at[b])
        pltpu.async_copy(x2_hbm.at[pl.ds(base, BW)], bx2, in_sems.at[b])

    def wait_in(b):
        bx0, bx1, bx2 = xbufs[b]
        pltpu.make_async_copy(x0_hbm.at[pl.ds(0, BW)], bx0,
                              in_sems.at[b]).wait()
        pltpu.make_async_copy(x1_hbm.at[pl.ds(0, BW)], bx1,
                              in_sems.at[b]).wait()
        pltpu.make_async_copy(x2_hbm.at[pl.ds(0, BW)], bx2,
                              in_sems.at[b]).wait()

    def step(t, b):
        issue_in(t, b)
        wait_in(b)
        bx0, bx1, bx2 = xbufs[b]

        @plsc.parallel_loop(0, NK, unroll=2)
        def idx_body(k):
            sl = pl.ds(k * 16, 16)
            idx1_v[sl] = (bx1[sl] * float(STEPS)).astype(jnp.int32)
            idx2_v[sl] = bx2[sl].astype(jnp.int32)

        def half_body(hb, _):
            hb0 = hb * HW
            hsl = pl.ds(b0 + hb0, HW)
            first = (t == 0) & (hb == 0)

            @pl.when(jnp.logical_not(first))
            def _():
                pltpu.make_async_copy(
                    feat_s, out_hbm.at[0, pl.ds(0, EMB), pl.ds(0, HW)],
                    out_sems.at[0]).wait()

            def feat_row(d, _2):
                d16 = jax.lax.broadcast_in_dim(d, (16,), ())
                wd16 = plsc.load_gather(wb_v, [d16])
                bd16 = plsc.load_gather(wb_v, [d16 + EMB])

                @plsc.parallel_loop(0, NK2, unroll=4)
                def feat_body(k):
                    sl = pl.ds(hb0 + k * 16, 16)
                    feat_s[d, pl.ds(k * 16, 16)] = bx0[sl] * wd16 + bd16
                return 0
            lax.fori_loop(0, EMB, feat_row, 0)
            pltpu.async_copy(
                feat_s, out_hbm.at[t, pl.ds(0, EMB), hsl], out_sems.at[0])

            @pl.when(jnp.logical_not(first))
            def _():
                pltpu.make_async_copy(
                    tod_s, out_hbm.at[0, pl.ds(0, EMB), pl.ds(0, HW)],
                    out_sems.at[1]).wait()

            @plsc.parallel_loop(0, NK2, unroll=1)
            def tod_body(k):
                idx1v = idx1_v[pl.ds(hb0 + k * 16, 16)]
                sl = pl.ds(k * 16, 16)
                for d in range(EMB):
                    tod_s[d, sl] = plsc.load_gather(
                        tod_t, [idx1v, jnp.full((16,), d, jnp.int32)])
            pltpu.async_copy(
                tod_s, out_hbm.at[t, pl.ds(EMB, EMB), hsl], out_sems.at[1])

            @pl.when(jnp.logical_not(first))
            def _():
                pltpu.make_async_copy(
                    dow_s, out_hbm.at[0, pl.ds(0, EMB), pl.ds(0, HW)],
                    out_sems.at[2]).wait()

            @plsc.parallel_loop(0, NK2, unroll=1)
            def dow_body(k):
                idx2v = idx2_v[pl.ds(hb0 + k * 16, 16)]
                sl = pl.ds(k * 16, 16)
                for d in range(EMB):
                    dow_s[d, sl] = plsc.load_gather(
                        dow_t, [idx2v, jnp.full((16,), d, jnp.int32)])
            pltpu.async_copy(
                dow_s, out_hbm.at[t, pl.ds(2 * EMB, EMB), hsl],
                out_sems.at[2])
            return 0

        lax.fori_loop(0, 2, half_body, 0)

    def pair_body(h, _):
        step(2 * h, 0)
        step(2 * h + 1, 1)
        return 0

    lax.fori_loop(0, T // NBUF, pair_body, 0)
    for sec_ref, s in ((feat_s, 0), (tod_s, 1), (dow_s, 2)):
        pltpu.make_async_copy(
            sec_ref, out_hbm.at[0, pl.ds(0, EMB), pl.ds(0, HW)],
            out_sems.at[s]).wait()


@jax.jit
def kernel(x, W_feat, b_feat, tod_table, dow_table):
    xT = jnp.transpose(x, (2, 1, 0))
    x0f = xT[0].reshape(T * B)
    x1f = xT[1].reshape(T * B)
    x2f = xT[2].reshape(T * B)
    mesh = plsc.VectorSubcoreMesh(core_axis_name="c", subcore_axis_name="s")
    f = pl.kernel(
        _sc_body,
        out_type=jax.ShapeDtypeStruct((T, 3 * EMB, B), jnp.float32),
        mesh=mesh,
        compiler_params=pltpu.CompilerParams(needs_layout_passes=False),
        scratch_types=[
            pltpu.VMEM((BW,), jnp.float32),
            pltpu.VMEM((BW,), jnp.float32),
            pltpu.VMEM((BW,), jnp.float32),
            pltpu.VMEM((BW,), jnp.float32),
            pltpu.VMEM((BW,), jnp.float32),
            pltpu.VMEM((BW,), jnp.float32),
            pltpu.VMEM((BW,), jnp.int32),
            pltpu.VMEM((BW,), jnp.int32),
            pltpu.VMEM((EMB, HW), jnp.float32),
            pltpu.VMEM((EMB, HW), jnp.float32),
            pltpu.VMEM((EMB, HW), jnp.float32),
            pltpu.VMEM((2 * EMB,), jnp.float32),
            pltpu.VMEM((STEPS, EMB), jnp.float32),
            pltpu.VMEM((8, EMB), jnp.float32),
            pltpu.SemaphoreType.DMA((NBUF,)),
            pltpu.SemaphoreType.DMA((3,)),
        ],
    )
    dow_pad = jnp.concatenate(
        [dow_table, jnp.zeros((1, EMB), jnp.float32)], axis=0)
    out = f(x0f, x1f, x2f, W_feat.reshape(EMB), b_feat, tod_table, dow_pad)
    return jnp.transpose(out, (2, 0, 1))
